# baseline (speedup 1.0000x reference)
import functools

import jax
import jax.numpy as jnp
from jax import lax
from jax.experimental import pallas as pl
from jax.experimental.pallas import tpu as pltpu
from jax.experimental.pallas import tpu_sc as plsc

N_GRID = 10000
N_GP = 10240
N_MESH = 2048
T = 8
GRU_H = 128
HID = 256
OUT_GRID = 64
N_PROC = 4

BN_EPS = 1e-5
LN_EPS = 1e-5


def _gru_call(xg, w_ihT, w_hhT, b_ih, b_hh, bn=1024):
    n = xg.shape[0]

    def body(x_ref, wi_ref, wh_ref, bi_ref, bh_ref, o_ref):
        h = jnp.zeros((bn, GRU_H), jnp.float32)
        wi = wi_ref[...]
        wh = wh_ref[...]
        bi = bi_ref[...]
        bh = bh_ref[...]
        for t in range(T):
            xt = x_ref[:, t * 10:(t + 1) * 10]
            gi = jnp.dot(xt, wi, preferred_element_type=jnp.float32) + bi
            gh = jnp.dot(h, wh, preferred_element_type=jnp.float32) + bh
            r = jax.nn.sigmoid(gi[:, 0:128] + gh[:, 0:128])
            z = jax.nn.sigmoid(gi[:, 128:256] + gh[:, 128:256])
            nn_ = jnp.tanh(gi[:, 256:384] + r * gh[:, 256:384])
            h = (1.0 - z) * nn_ + z * h
        o_ref[...] = h

    return pl.pallas_call(
        body,
        grid=(n // bn,),
        in_specs=[
            pl.BlockSpec((bn, T * 10), lambda i: (i, 0)),
            pl.BlockSpec((10, 384), lambda i: (0, 0)),
            pl.BlockSpec((GRU_H, 384), lambda i: (0, 0)),
            pl.BlockSpec((384,), lambda i: (0,)),
            pl.BlockSpec((384,), lambda i: (0,)),
        ],
        out_specs=pl.BlockSpec((bn, GRU_H), lambda i: (i, 0)),
        out_shape=jax.ShapeDtypeStruct((n, GRU_H), jnp.float32),
    )(xg, w_ihT, w_hhT, b_ih, b_hh)


def _bn_concat_call(h, x_spa, bn_g, bn_b, spa_g, spa_b):
    def body(h_ref, s_ref, g_ref, b_ref, sg_ref, sb_ref, o_ref):
        rows = lax.broadcasted_iota(jnp.int32, (N_GP, 1), 0)
        mask = (rows < N_GRID).astype(jnp.float32)
        inv_n = 1.0 / N_GRID
        h = h_ref[...]
        hm = h * mask
        mu = jnp.sum(hm, axis=0, keepdims=True) * inv_n
        d = (h - mu) * mask
        var = jnp.sum(d * d, axis=0, keepdims=True) * inv_n
        hn = (h - mu) * jax.lax.rsqrt(var + BN_EPS) * g_ref[...] + b_ref[...]
        s = s_ref[...]
        sm = s * mask
        smu = jnp.sum(sm, axis=0, keepdims=True) * inv_n
        sd = (s - smu) * mask
        svar = jnp.sum(sd * sd, axis=0, keepdims=True) * inv_n
        sn = (s - smu) * jax.lax.rsqrt(svar + BN_EPS) * sg_ref[...] + sb_ref[...]
        o_ref[:, 0:GRU_H] = hn
        o_ref[:, GRU_H:GRU_H + 4] = sn

    return pl.pallas_call(
        body,
        out_shape=jax.ShapeDtypeStruct((N_GP, GRU_H + 4), jnp.float32),
    )(h, x_spa, bn_g.reshape(1, -1), bn_b.reshape(1, -1),
      spa_g.reshape(1, -1), spa_b.reshape(1, -1))


def _mlp2_call(x, w1, b1, w2, b2, ln=None, split_out=False, bn=None):
    n, k = x.shape
    dh = w1.shape[1]
    dout = w2.shape[1]
    if bn is None:
        bn = n if n <= 4096 else 1024
    assert n % bn == 0
    nsplit = dout // 128
    has_ln = ln is not None

    def body(x_ref, w1_ref, b1_ref, w2_ref, b2_ref, *rest):
        if has_ln:
            g_ref, be_ref, o_ref = rest
        else:
            (o_ref,) = rest
        h = jnp.maximum(
            jnp.dot(x_ref[...], w1_ref[...],
                    preferred_element_type=jnp.float32) + b1_ref[...], 0.0)
        y = jnp.dot(h, w2_ref[...], preferred_element_type=jnp.float32) + b2_ref[...]
        if has_ln:
            mu = jnp.mean(y, axis=-1, keepdims=True)
            d = y - mu
            var = jnp.mean(d * d, axis=-1, keepdims=True)
            y = d * jax.lax.rsqrt(var + LN_EPS) * g_ref[...] + be_ref[...]
        if split_out:
            for j in range(nsplit):
                o_ref[j, :, :] = y[:, 128 * j:128 * (j + 1)]
        else:
            o_ref[...] = y

    in_specs = [
        pl.BlockSpec((bn, k), lambda i: (i, 0)),
        pl.BlockSpec((k, dh), lambda i: (0, 0)),
        pl.BlockSpec((1, dh), lambda i: (0, 0)),
        pl.BlockSpec((dh, dout), lambda i: (0, 0)),
        pl.BlockSpec((1, dout), lambda i: (0, 0)),
    ]
    args = [x, w1, b1.reshape(1, -1), w2, b2.reshape(1, -1)]
    if has_ln:
        in_specs += [pl.BlockSpec((1, dout), lambda i: (0, 0)),
                     pl.BlockSpec((1, dout), lambda i: (0, 0))]
        args += [ln[0].reshape(1, -1), ln[1].reshape(1, -1)]
    if split_out:
        out_specs = pl.BlockSpec((nsplit, bn, 128), lambda i: (0, i, 0))
        out_shape = jax.ShapeDtypeStruct((nsplit, n, 128), jnp.float32)
    else:
        out_specs = pl.BlockSpec((bn, dout), lambda i: (i, 0))
        out_shape = jax.ShapeDtypeStruct((n, dout), jnp.float32)
    return pl.pallas_call(
        body, grid=(n // bn,), in_specs=in_specs,
        out_specs=out_specs, out_shape=out_shape)(*args)


def _matmul_split_call(x, w, bn=None):
    n, k = x.shape
    dout = w.shape[1]
    nsplit = dout // 128
    if bn is None:
        bn = n if n <= 4096 else 1024
    assert n % bn == 0

    def body(x_ref, w_ref, o_ref):
        y = jnp.dot(x_ref[...], w_ref[...], preferred_element_type=jnp.float32)
        for j in range(nsplit):
            o_ref[j, :, :] = y[:, 128 * j:128 * (j + 1)]

    return pl.pallas_call(
        body, grid=(n // bn,),
        in_specs=[pl.BlockSpec((bn, k), lambda i: (i, 0)),
                  pl.BlockSpec((k, dout), lambda i: (0, 0))],
        out_specs=pl.BlockSpec((nsplit, bn, 128), lambda i: (0, i, 0)),
        out_shape=jax.ShapeDtypeStruct((nsplit, n, 128), jnp.float32))(x, w)


def _post_stage_call(acc, prev, w2, b2, ln=None, bn=None):
    n = prev.shape[0]
    if bn is None:
        bn = n if n <= 4096 else 1024
    assert n % bn == 0
    has_ln = ln is not None

    def body(a_ref, p_ref, w_ref, b_ref, *rest):
        if has_ln:
            g_ref, be_ref, o_ref = rest
        else:
            (o_ref,) = rest
        a0 = a_ref[0, :, 0:128]
        a1 = a_ref[1, :, 0:128]
        cnt = a_ref[0, :, 128:129]
        y = (p_ref[...]
             + jnp.dot(a0, w_ref[0:128, :], preferred_element_type=jnp.float32)
             + jnp.dot(a1, w_ref[128:256, :], preferred_element_type=jnp.float32)
             + cnt * b_ref[...])
        if has_ln:
            mu = jnp.mean(y, axis=-1, keepdims=True)
            d = y - mu
            var = jnp.mean(d * d, axis=-1, keepdims=True)
            y = d * jax.lax.rsqrt(var + LN_EPS) * g_ref[...] + be_ref[...]
        o_ref[...] = y

    in_specs = [
        pl.BlockSpec((2, bn, 144), lambda i: (0, i, 0)),
        pl.BlockSpec((bn, HID), lambda i: (i, 0)),
        pl.BlockSpec((HID, HID), lambda i: (0, 0)),
        pl.BlockSpec((1, HID), lambda i: (0, 0)),
    ]
    args = [acc, prev, w2, b2.reshape(1, -1)]
    if has_ln:
        in_specs += [pl.BlockSpec((1, HID), lambda i: (0, 0)),
                     pl.BlockSpec((1, HID), lambda i: (0, 0))]
        args += [ln[0].reshape(1, -1), ln[1].reshape(1, -1)]
    return pl.pallas_call(
        body, grid=(n // bn,), in_specs=in_specs,
        out_specs=pl.BlockSpec((bn, HID), lambda i: (i, 0)),
        out_shape=jax.ShapeDtypeStruct((n, HID), jnp.float32))(*args)


def _decoder_call(x, p, bn=1024):
    n = x.shape[0]
    (d1w, d1b), (d2w, d2b) = p['dec']
    l1w, l1b = p['lin1']
    l2w, l2b = p['lin2']
    ow, ob = p['out']

    def body(x_ref, d1w_r, d1b_r, d2w_r, d2b_r, l1w_r, l1b_r, l2w_r, l2b_r,
             ow_r, ob_r, out_r, log_r, hid_r):
        h = jnp.maximum(jnp.dot(x_ref[...], d1w_r[...],
                                preferred_element_type=jnp.float32) + d1b_r[...], 0.0)
        y = jnp.dot(h, d2w_r[...], preferred_element_type=jnp.float32) + d2b_r[...]
        y = jnp.maximum(jnp.dot(y, l1w_r[...],
                                preferred_element_type=jnp.float32) + l1b_r[...], 0.0)
        hid = jnp.maximum(jnp.dot(y, l2w_r[...],
                                  preferred_element_type=jnp.float32) + l2b_r[...], 0.0)
        logits = jnp.dot(hid, ow_r[...], preferred_element_type=jnp.float32) + ob_r[...]
        cmask = (lax.broadcasted_iota(jnp.int32, (1, 8), 1) < 4).astype(jnp.float32)
        m = jnp.max(logits - 1e30 * (1.0 - cmask), axis=-1, keepdims=True)
        e = jnp.exp(logits - m) * cmask
        out_r[...] = e / jnp.sum(e, axis=-1, keepdims=True)
        log_r[...] = logits
        hid_r[...] = hid

    wspec = lambda shape: pl.BlockSpec(shape, lambda i: tuple(0 for _ in shape))
    return pl.pallas_call(
        body, grid=(n // bn,),
        in_specs=[
            pl.BlockSpec((bn, HID), lambda i: (i, 0)),
            wspec((HID, HID)), wspec((1, HID)),
            wspec((HID, OUT_GRID)), wspec((1, OUT_GRID)),
            wspec((OUT_GRID, 64)), wspec((1, 64)),
            wspec((64, 64)), wspec((1, 64)),
            wspec((64, 8)), wspec((1, 8)),
        ],
        out_specs=[
            pl.BlockSpec((bn, 8), lambda i: (i, 0)),
            pl.BlockSpec((bn, 8), lambda i: (i, 0)),
            pl.BlockSpec((bn, 64), lambda i: (i, 0)),
        ],
        out_shape=[
            jax.ShapeDtypeStruct((n, 8), jnp.float32),
            jax.ShapeDtypeStruct((n, 8), jnp.float32),
            jax.ShapeDtypeStruct((n, 64), jnp.float32),
        ],
    )(x, d1w, d1b.reshape(1, -1), d2w, d2b.reshape(1, -1),
      l1w, l1b.reshape(1, -1), l2w, l2b.reshape(1, -1),
      jnp.pad(ow, ((0, 0), (0, 4))), jnp.pad(ob, (0, 4)).reshape(1, -1))


def _edge_stage(src_tab, dst_tab, e_p, src_idx, dst_idx, n_acc):
    ns = src_tab.shape[0] // 2
    nd = dst_tab.shape[0] // 2
    accs = []
    for c in range(2):
        h = jnp.maximum(src_tab[src_idx + c * ns] + dst_tab[dst_idx + c * nd]
                        + e_p[c], 0.0)
        a = jax.ops.segment_sum(h, dst_idx, n_acc)
        accs.append(jnp.pad(a, ((0, 0), (0, 16))))
    acc = jnp.stack(accs)
    cnt = jax.ops.segment_sum(jnp.ones((src_idx.shape[0],), jnp.float32),
                              dst_idx, n_acc)
    acc = acc.at[0, :, 128].set(cnt)
    return acc


def kernel(X, mesh_feat, mesh_ei, g2m_src, g2m_dst, m2g_src, m2g_dst,
           e_mm, e_g2m, e_m2g, params):
    p = params
    f32 = jnp.float32

    Xt = jnp.transpose(X[:, 0:10, :], (0, 2, 1)).reshape(N_GRID, T * 10)
    Xt = jnp.pad(Xt, ((0, N_GP - N_GRID), (0, 0)))
    X_spa = jnp.pad(X[:, 10:14, -1], ((0, N_GP - N_GRID), (0, 0)))

    w_ihT = p['gru_W_ih'].T.astype(f32)
    w_hhT = p['gru_W_hh'].T.astype(f32)

    def msg_split(w):
        return w[0:HID], w[HID:2 * HID], w[2 * HID:3 * HID]

    (g2m_w1, g2m_b1), (g2m_w2, g2m_b2) = p['g2m_msg']
    g2m_w1s, g2m_w1d, g2m_w1e = msg_split(g2m_w1)
    (m2g_w1, m2g_b1), (m2g_w2, m2g_b2) = p['m2g_msg']
    m2g_w1s, m2g_w1d, m2g_w1e = msg_split(m2g_w1)
    proc_w1s, proc_w1d, proc_w1e, proc_w2, proc_b2, proc_b1 = [], [], [], [], [], []
    for l in range(N_PROC):
        (w1, b1), (w2, b2) = p['proc'][l]
        s, d, e = msg_split(w1)
        proc_w1s.append(s); proc_w1d.append(d); proc_w1e.append(e)
        proc_w2.append(w2); proc_b2.append(b2); proc_b1.append(b1)

    (eg_w1, eg_b1), (eg_w2, eg_b2) = p['eg2m_enc']
    eg_w2f = eg_w2 @ g2m_w1e
    eg_b2f = eg_b2 @ g2m_w1e + g2m_b1
    (em_w1, em_b1), (em_w2, em_b2) = p['emm_enc']
    em_w2f = jnp.concatenate([em_w2 @ proc_w1e[l] for l in range(N_PROC)], axis=1)
    em_b2f = jnp.concatenate([em_b2 @ proc_w1e[l] + proc_b1[l]
                              for l in range(N_PROC)])
    (e2_w1, e2_b1), (e2_w2, e2_b2) = p['em2g_enc']
    e2_w2f = e2_w2 @ m2g_w1e
    e2_b2f = e2_b2 @ m2g_w1e + m2g_b1

    def prep_edges(src, dst, e_feat, dummy_dst):
        E = src.shape[0]
        Ep = ((E + 2047) // 2048) * 2048
        src = jnp.pad(src.astype(jnp.int32), (0, Ep - E))
        dst = jnp.pad(dst.astype(jnp.int32), (0, Ep - E),
                      constant_values=dummy_dst)
        e_feat = jnp.pad(e_feat.astype(f32), ((0, Ep - E), (0, 0)))
        return src, dst, e_feat

    g2m_s, g2m_d, e_g2m_p = prep_edges(g2m_src, g2m_dst, e_g2m, N_MESH)
    mm_s, mm_d, e_mm_p = prep_edges(mesh_ei[0], mesh_ei[1], e_mm, N_MESH)
    m2g_s, m2g_d, e_m2g_p = prep_edges(m2g_src, m2g_dst, e_m2g, N_GRID)

    N_ACC_M = N_MESH + 8
    N_ACC_G = N_GP

    h = _gru_call(Xt, w_ihT, w_hhT, p['gru_b_ih'], p['gru_b_hh'])
    xgc = _bn_concat_call(h, X_spa, p['bn_g'], p['bn_b'], p['spa_g'], p['spa_b'])

    (gw1, gb1), (gw2, gb2) = p['grid_enc']
    grid = _mlp2_call(xgc, jnp.pad(gw1, ((0, 0), (0, 0))), gb1, gw2, gb2,
                      ln=p['grid_ln'])
    (mw1, mb1), (mw2, mb2) = p['mesh_enc']
    mesh = _mlp2_call(mesh_feat.astype(f32), mw1, mb1, mw2, mb2,
                      ln=p['mesh_ln'])

    eg2m_p = _mlp2_call(e_g2m_p, eg_w1, eg_b1, eg_w2f, eg_b2f, split_out=True)
    emm_p = _mlp2_call(e_mm_p, em_w1, em_b1, em_w2f, em_b2f, split_out=True)
    em2g_p = _mlp2_call(e_m2g_p, e2_w1, e2_b1, e2_w2f, e2_b2f, split_out=True)

    gsrc_tab = _matmul_split_call(grid, g2m_w1s).reshape(2 * N_GP, 128)
    mdst_tab = _matmul_split_call(mesh, g2m_w1d).reshape(2 * N_MESH, 128)
    acc = _edge_stage(gsrc_tab, mdst_tab, eg2m_p, g2m_s, g2m_d, N_ACC_M)
    mesh = _post_stage_call(acc[:, 0:N_MESH], mesh, g2m_w2, g2m_b2)

    for l in range(N_PROC):
        sd = _matmul_split_call(
            mesh, jnp.concatenate([proc_w1s[l], proc_w1d[l]], axis=1))
        s_tab = sd[0:2].reshape(2 * N_MESH, 128)
        d_tab = sd[2:4].reshape(2 * N_MESH, 128)
        acc = _edge_stage(s_tab, d_tab, emm_p[2 * l:2 * l + 2], mm_s, mm_d,
                          N_ACC_M)
        mesh = _post_stage_call(acc[:, 0:N_MESH], mesh, proc_w2[l], proc_b2[l],
                                ln=p['proc_ln'][l])

    msrc_tab = _matmul_split_call(mesh, m2g_w1s).reshape(2 * N_MESH, 128)
    gdst_tab = _matmul_split_call(grid, m2g_w1d).reshape(2 * N_GP, 128)
    acc = _edge_stage(msrc_tab, gdst_tab, em2g_p, m2g_s, m2g_d, N_ACC_G)
    grid = _post_stage_call(acc, grid, m2g_w2, m2g_b2)

    output, logits, hidden = _decoder_call(grid, p)
    return (output[0:N_GRID, 0:4], logits[0:N_GRID, 0:4], hidden[0:N_GRID])


# speedup vs baseline: 3.8793x; 3.8793x over previous
import functools

import jax
import jax.numpy as jnp
from jax import lax
from jax.experimental import pallas as pl
from jax.experimental.pallas import tpu as pltpu
from jax.experimental.pallas import tpu_sc as plsc

N_GRID = 10000
N_GP = 10240
N_MESH = 2048
T = 8
GRU_H = 128
HID = 256
OUT_GRID = 64
N_PROC = 4

BN_EPS = 1e-5
LN_EPS = 1e-5


def _gru_call(xg, w_ihT, w_hhT, b_ih, b_hh, bn=1024):
    n = xg.shape[0]

    def body(x_ref, wi_ref, wh_ref, bi_ref, bh_ref, o_ref):
        h = jnp.zeros((bn, GRU_H), jnp.float32)
        wi = wi_ref[...]
        wh = wh_ref[...]
        bi = bi_ref[...]
        bh = bh_ref[...]
        for t in range(T):
            xt = x_ref[:, t * 10:(t + 1) * 10]
            gi = jnp.dot(xt, wi, preferred_element_type=jnp.float32) + bi
            gh = jnp.dot(h, wh, preferred_element_type=jnp.float32) + bh
            r = jax.nn.sigmoid(gi[:, 0:128] + gh[:, 0:128])
            z = jax.nn.sigmoid(gi[:, 128:256] + gh[:, 128:256])
            nn_ = jnp.tanh(gi[:, 256:384] + r * gh[:, 256:384])
            h = (1.0 - z) * nn_ + z * h
        o_ref[...] = h

    return pl.pallas_call(
        body,
        grid=(n // bn,),
        in_specs=[
            pl.BlockSpec((bn, T * 10), lambda i: (i, 0)),
            pl.BlockSpec((10, 384), lambda i: (0, 0)),
            pl.BlockSpec((GRU_H, 384), lambda i: (0, 0)),
            pl.BlockSpec((384,), lambda i: (0,)),
            pl.BlockSpec((384,), lambda i: (0,)),
        ],
        out_specs=pl.BlockSpec((bn, GRU_H), lambda i: (i, 0)),
        out_shape=jax.ShapeDtypeStruct((n, GRU_H), jnp.float32),
    )(xg, w_ihT, w_hhT, b_ih, b_hh)


def _bn_concat_call(h, x_spa, bn_g, bn_b, spa_g, spa_b):
    def body(h_ref, s_ref, g_ref, b_ref, sg_ref, sb_ref, o_ref):
        rows = lax.broadcasted_iota(jnp.int32, (N_GP, 1), 0)
        mask = (rows < N_GRID).astype(jnp.float32)
        inv_n = 1.0 / N_GRID
        h = h_ref[...]
        hm = h * mask
        mu = jnp.sum(hm, axis=0, keepdims=True) * inv_n
        d = (h - mu) * mask
        var = jnp.sum(d * d, axis=0, keepdims=True) * inv_n
        hn = (h - mu) * jax.lax.rsqrt(var + BN_EPS) * g_ref[...] + b_ref[...]
        s = s_ref[...]
        sm = s * mask
        smu = jnp.sum(sm, axis=0, keepdims=True) * inv_n
        sd = (s - smu) * mask
        svar = jnp.sum(sd * sd, axis=0, keepdims=True) * inv_n
        sn = (s - smu) * jax.lax.rsqrt(svar + BN_EPS) * sg_ref[...] + sb_ref[...]
        o_ref[:, 0:GRU_H] = hn
        o_ref[:, GRU_H:GRU_H + 4] = sn

    return pl.pallas_call(
        body,
        out_shape=jax.ShapeDtypeStruct((N_GP, GRU_H + 4), jnp.float32),
    )(h, x_spa, bn_g.reshape(1, -1), bn_b.reshape(1, -1),
      spa_g.reshape(1, -1), spa_b.reshape(1, -1))


def _mlp2_call(x, w1, b1, w2, b2, ln=None, split_out=False, bn=None):
    n, k = x.shape
    dh = w1.shape[1]
    dout = w2.shape[1]
    if bn is None:
        bn = n if n <= 4096 else 1024
    assert n % bn == 0
    nsplit = dout // 128
    has_ln = ln is not None

    def body(x_ref, w1_ref, b1_ref, w2_ref, b2_ref, *rest):
        if has_ln:
            g_ref, be_ref, o_ref = rest
        else:
            (o_ref,) = rest
        h = jnp.maximum(
            jnp.dot(x_ref[...], w1_ref[...],
                    preferred_element_type=jnp.float32) + b1_ref[...], 0.0)
        y = jnp.dot(h, w2_ref[...], preferred_element_type=jnp.float32) + b2_ref[...]
        if has_ln:
            mu = jnp.mean(y, axis=-1, keepdims=True)
            d = y - mu
            var = jnp.mean(d * d, axis=-1, keepdims=True)
            y = d * jax.lax.rsqrt(var + LN_EPS) * g_ref[...] + be_ref[...]
        if split_out:
            for j in range(nsplit):
                o_ref[j, :, :] = y[:, 128 * j:128 * (j + 1)]
        else:
            o_ref[...] = y

    in_specs = [
        pl.BlockSpec((bn, k), lambda i: (i, 0)),
        pl.BlockSpec((k, dh), lambda i: (0, 0)),
        pl.BlockSpec((1, dh), lambda i: (0, 0)),
        pl.BlockSpec((dh, dout), lambda i: (0, 0)),
        pl.BlockSpec((1, dout), lambda i: (0, 0)),
    ]
    args = [x, w1, b1.reshape(1, -1), w2, b2.reshape(1, -1)]
    if has_ln:
        in_specs += [pl.BlockSpec((1, dout), lambda i: (0, 0)),
                     pl.BlockSpec((1, dout), lambda i: (0, 0))]
        args += [ln[0].reshape(1, -1), ln[1].reshape(1, -1)]
    if split_out:
        out_specs = pl.BlockSpec((nsplit, bn, 128), lambda i: (0, i, 0))
        out_shape = jax.ShapeDtypeStruct((nsplit, n, 128), jnp.float32)
    else:
        out_specs = pl.BlockSpec((bn, dout), lambda i: (i, 0))
        out_shape = jax.ShapeDtypeStruct((n, dout), jnp.float32)
    return pl.pallas_call(
        body, grid=(n // bn,), in_specs=in_specs,
        out_specs=out_specs, out_shape=out_shape)(*args)


def _matmul_split_call(x, w, bn=None):
    n, k = x.shape
    dout = w.shape[1]
    nsplit = dout // 128
    if bn is None:
        bn = n if n <= 4096 else 1024
    assert n % bn == 0

    def body(x_ref, w_ref, o_ref):
        y = jnp.dot(x_ref[...], w_ref[...], preferred_element_type=jnp.float32)
        for j in range(nsplit):
            o_ref[j, :, :] = y[:, 128 * j:128 * (j + 1)]

    return pl.pallas_call(
        body, grid=(n // bn,),
        in_specs=[pl.BlockSpec((bn, k), lambda i: (i, 0)),
                  pl.BlockSpec((k, dout), lambda i: (0, 0))],
        out_specs=pl.BlockSpec((nsplit, bn, 128), lambda i: (0, i, 0)),
        out_shape=jax.ShapeDtypeStruct((nsplit, n, 128), jnp.float32))(x, w)


def _post_stage_call(acc, cnt, prev, w2, b2, ln=None, bn=None):
    n = prev.shape[0]
    if bn is None:
        bn = n if n <= 4096 else 1024
    assert n % bn == 0
    has_ln = ln is not None

    def body(a_ref, c_ref, p_ref, w_ref, b_ref, *rest):
        if has_ln:
            g_ref, be_ref, o_ref = rest
        else:
            (o_ref,) = rest
        a0 = a_ref[0, :, :]
        a1 = a_ref[1, :, :]
        cnt_col = c_ref[0, :, 0:1] + c_ref[1, :, 0:1]
        y = (p_ref[...]
             + jnp.dot(a0, w_ref[0:128, :], preferred_element_type=jnp.float32)
             + jnp.dot(a1, w_ref[128:256, :], preferred_element_type=jnp.float32)
             + cnt_col * b_ref[...])
        if has_ln:
            mu = jnp.mean(y, axis=-1, keepdims=True)
            d = y - mu
            var = jnp.mean(d * d, axis=-1, keepdims=True)
            y = d * jax.lax.rsqrt(var + LN_EPS) * g_ref[...] + be_ref[...]
        o_ref[...] = y

    in_specs = [
        pl.BlockSpec((2, bn, 128), lambda i: (0, i, 0)),
        pl.BlockSpec((2, bn, 128), lambda i: (0, i, 0)),
        pl.BlockSpec((bn, HID), lambda i: (i, 0)),
        pl.BlockSpec((HID, HID), lambda i: (0, 0)),
        pl.BlockSpec((1, HID), lambda i: (0, 0)),
    ]
    args = [acc, cnt, prev, w2, b2.reshape(1, -1)]
    if has_ln:
        in_specs += [pl.BlockSpec((1, HID), lambda i: (0, 0)),
                     pl.BlockSpec((1, HID), lambda i: (0, 0))]
        args += [ln[0].reshape(1, -1), ln[1].reshape(1, -1)]
    return pl.pallas_call(
        body, grid=(n // bn,), in_specs=in_specs,
        out_specs=pl.BlockSpec((bn, HID), lambda i: (i, 0)),
        out_shape=jax.ShapeDtypeStruct((n, HID), jnp.float32))(*args)


def _decoder_call(x, p, bn=1024):
    n = x.shape[0]
    (d1w, d1b), (d2w, d2b) = p['dec']
    l1w, l1b = p['lin1']
    l2w, l2b = p['lin2']
    ow, ob = p['out']

    def body(x_ref, d1w_r, d1b_r, d2w_r, d2b_r, l1w_r, l1b_r, l2w_r, l2b_r,
             ow_r, ob_r, out_r, log_r, hid_r):
        h = jnp.maximum(jnp.dot(x_ref[...], d1w_r[...],
                                preferred_element_type=jnp.float32) + d1b_r[...], 0.0)
        y = jnp.dot(h, d2w_r[...], preferred_element_type=jnp.float32) + d2b_r[...]
        y = jnp.maximum(jnp.dot(y, l1w_r[...],
                                preferred_element_type=jnp.float32) + l1b_r[...], 0.0)
        hid = jnp.maximum(jnp.dot(y, l2w_r[...],
                                  preferred_element_type=jnp.float32) + l2b_r[...], 0.0)
        logits = jnp.dot(hid, ow_r[...], preferred_element_type=jnp.float32) + ob_r[...]
        cmask = (lax.broadcasted_iota(jnp.int32, (1, 8), 1) < 4).astype(jnp.float32)
        m = jnp.max(logits - 1e30 * (1.0 - cmask), axis=-1, keepdims=True)
        e = jnp.exp(logits - m) * cmask
        out_r[...] = e / jnp.sum(e, axis=-1, keepdims=True)
        log_r[...] = logits
        hid_r[...] = hid

    wspec = lambda shape: pl.BlockSpec(shape, lambda i: tuple(0 for _ in shape))
    return pl.pallas_call(
        body, grid=(n // bn,),
        in_specs=[
            pl.BlockSpec((bn, HID), lambda i: (i, 0)),
            wspec((HID, HID)), wspec((1, HID)),
            wspec((HID, OUT_GRID)), wspec((1, OUT_GRID)),
            wspec((OUT_GRID, 64)), wspec((1, 64)),
            wspec((64, 64)), wspec((1, 64)),
            wspec((64, 8)), wspec((1, 8)),
        ],
        out_specs=[
            pl.BlockSpec((bn, 8), lambda i: (i, 0)),
            pl.BlockSpec((bn, 8), lambda i: (i, 0)),
            pl.BlockSpec((bn, 64), lambda i: (i, 0)),
        ],
        out_shape=[
            jax.ShapeDtypeStruct((n, 8), jnp.float32),
            jax.ShapeDtypeStruct((n, 8), jnp.float32),
            jax.ShapeDtypeStruct((n, 64), jnp.float32),
        ],
    )(x, d1w, d1b.reshape(1, -1), d2w, d2b.reshape(1, -1),
      l1w, l1b.reshape(1, -1), l2w, l2b.reshape(1, -1),
      jnp.pad(ow, ((0, 0), (0, 4))), jnp.pad(ob, (0, 4)).reshape(1, -1))


_NS_T = 16
_CH = 64


def _edge_stage_sc(src_tab, dst_tab, e_p, sgi, dgi, dsi, zeros, n_acc_sp,
                   n_out):
    e_pad = e_p.shape[1]
    e_per_tile = e_pad // _NS_T
    n_chunks = e_per_tile // _CH
    rpt_z = n_acc_sp // _NS_T
    rpt_o = n_out // _NS_T
    mesh = plsc.VectorSubcoreMesh(core_axis_name="c", subcore_axis_name="s")

    @functools.partial(
        pl.kernel,
        out_type=jax.ShapeDtypeStruct((2, n_out, 128), jnp.float32),
        mesh=mesh,
        scratch_types=[
            pltpu.VMEM((n_chunks, _CH), jnp.int32),
            pltpu.VMEM((n_chunks, _CH), jnp.int32),
            pltpu.VMEM((n_chunks, _CH), jnp.int32),
            pltpu.VMEM((_CH, 128), jnp.float32),
            pltpu.VMEM((_CH, 128), jnp.float32),
            pltpu.VMEM((_CH, 128), jnp.float32),
            pltpu.VMEM((_CH, 128), jnp.float32),
            pltpu.VMEM_SHARED((n_acc_sp, 128), jnp.float32),
            pltpu.SemaphoreType.DMA,
            pltpu.SemaphoreType.DMA,
        ],
    )
    def k(src_tab_h, dst_tab_h, e_p_h, sgi_h, dgi_h, dsi_h, zeros_h, out_h,
          sidx, didx, dsidx, srow, drow, erow, msg, acc, sem1, sem2):
        c = lax.axis_index("c")
        s = lax.axis_index("s")
        base = s * e_per_tile

        pltpu.sync_copy(sgi_h.at[c, s], sidx)
        pltpu.sync_copy(dgi_h.at[c, s], didx)
        pltpu.sync_copy(dsi_h.at[s], dsidx)
        pltpu.sync_copy(zeros_h.at[pl.ds(0, rpt_z)],
                        acc.at[pl.ds(s * rpt_z, rpt_z)])
        plsc.subcore_barrier()

        def chunk(j, _):
            d1 = pltpu.async_copy(src_tab_h.at[sidx.at[j]], srow, sem1)
            d2 = pltpu.async_copy(dst_tab_h.at[didx.at[j]], drow, sem2)
            pltpu.sync_copy(e_p_h.at[c, pl.ds(base + j * _CH, _CH)], erow)
            d1.wait()
            d2.wait()

            def row(i, _):
                for jj in range(8):
                    sl = pl.ds(16 * jj, 16)
                    msg[i, sl] = jnp.maximum(
                        srow[i, sl] + drow[i, sl] + erow[i, sl], 0.0)
                return _

            lax.fori_loop(0, _CH, row, None)
            pltpu.sync_copy(msg, acc.at[dsidx.at[j]], add=True)
            return _

        lax.fori_loop(0, n_chunks, chunk, None)
        plsc.subcore_barrier()
        pltpu.sync_copy(acc.at[pl.ds(s * rpt_o, rpt_o)],
                        out_h.at[c, pl.ds(s * rpt_o, rpt_o)])

    return k(src_tab, dst_tab, e_p, sgi, dgi, dsi, zeros)


def _counts_sc(dsi_list, aug, zeros, n_accs, n_outs):
    n_sp = max(n_accs)
    mesh = plsc.VectorSubcoreMesh(core_axis_name="c", subcore_axis_name="s")
    max_nch = max(d.shape[2] for d in dsi_list)

    @functools.partial(
        pl.kernel,
        out_type=tuple(jax.ShapeDtypeStruct((2, n, 128), jnp.float32)
                       for n in n_outs),
        mesh=mesh,
        scratch_types=[
            pltpu.VMEM((max_nch, _CH), jnp.int32),
            pltpu.VMEM((_CH, 128), jnp.float32),
            pltpu.VMEM_SHARED((n_sp, 128), jnp.float32),
        ],
    )
    def k(*refs):
        nl = len(dsi_list)
        dsis = refs[0:nl]
        aug_h, zeros_h = refs[nl], refs[nl + 1]
        outs = refs[nl + 2:2 * nl + 2]
        dsidx, augv, acc = refs[2 * nl + 2:]
        c = lax.axis_index("c")
        s = lax.axis_index("s")
        pltpu.sync_copy(aug_h, augv)
        for l in range(nl):
            n_acc, n_out = n_accs[l], n_outs[l]
            nch = dsi_list[l].shape[2]
            rpt_z = n_acc // _NS_T
            rpt_o = n_out // _NS_T
            pltpu.sync_copy(zeros_h.at[pl.ds(0, rpt_z)],
                            acc.at[pl.ds(s * rpt_z, rpt_z)])
            plsc.subcore_barrier()
            pltpu.sync_copy(dsis[l].at[c, s], dsidx.at[pl.ds(0, nch)])

            def chunk(j, _):
                pltpu.sync_copy(augv, acc.at[dsidx.at[j]], add=True)
                return _

            lax.fori_loop(0, nch, chunk, None)
            plsc.subcore_barrier()
            pltpu.sync_copy(acc.at[pl.ds(s * rpt_o, rpt_o)],
                            outs[l].at[c, pl.ds(s * rpt_o, rpt_o)])
            plsc.subcore_barrier()

    return k(*dsi_list, aug, zeros)


def kernel(X, mesh_feat, mesh_ei, g2m_src, g2m_dst, m2g_src, m2g_dst,
           e_mm, e_g2m, e_m2g, params):
    p = params
    f32 = jnp.float32

    Xt = jnp.transpose(X[:, 0:10, :], (0, 2, 1)).reshape(N_GRID, T * 10)
    Xt = jnp.pad(Xt, ((0, N_GP - N_GRID), (0, 0)))
    X_spa = jnp.pad(X[:, 10:14, -1], ((0, N_GP - N_GRID), (0, 0)))

    w_ihT = p['gru_W_ih'].T.astype(f32)
    w_hhT = p['gru_W_hh'].T.astype(f32)

    def msg_split(w):
        return w[0:HID], w[HID:2 * HID], w[2 * HID:3 * HID]

    (g2m_w1, g2m_b1), (g2m_w2, g2m_b2) = p['g2m_msg']
    g2m_w1s, g2m_w1d, g2m_w1e = msg_split(g2m_w1)
    (m2g_w1, m2g_b1), (m2g_w2, m2g_b2) = p['m2g_msg']
    m2g_w1s, m2g_w1d, m2g_w1e = msg_split(m2g_w1)
    proc_w1s, proc_w1d, proc_w1e, proc_w2, proc_b2, proc_b1 = [], [], [], [], [], []
    for l in range(N_PROC):
        (w1, b1), (w2, b2) = p['proc'][l]
        s, d, e = msg_split(w1)
        proc_w1s.append(s); proc_w1d.append(d); proc_w1e.append(e)
        proc_w2.append(w2); proc_b2.append(b2); proc_b1.append(b1)

    (eg_w1, eg_b1), (eg_w2, eg_b2) = p['eg2m_enc']
    eg_w2f = eg_w2 @ g2m_w1e
    eg_b2f = eg_b2 @ g2m_w1e + g2m_b1
    (em_w1, em_b1), (em_w2, em_b2) = p['emm_enc']
    em_w2f = jnp.concatenate([em_w2 @ proc_w1e[l] for l in range(N_PROC)], axis=1)
    em_b2f = jnp.concatenate([em_b2 @ proc_w1e[l] + proc_b1[l]
                              for l in range(N_PROC)])
    (e2_w1, e2_b1), (e2_w2, e2_b2) = p['em2g_enc']
    e2_w2f = e2_w2 @ m2g_w1e
    e2_b2f = e2_b2 @ m2g_w1e + m2g_b1

    def prep_edges(src, dst, e_feat, dummy_dst, n_src_tab, n_dst_tab):
        E = src.shape[0]
        Ep = ((E + 2047) // 2048) * 2048
        src = jnp.pad(src.astype(jnp.int32), (0, Ep - E))
        dst = jnp.pad(dst.astype(jnp.int32), (0, Ep - E),
                      constant_values=dummy_dst)
        e_feat = jnp.pad(e_feat.astype(f32), ((0, Ep - E), (0, 0)))
        nch = Ep // (_NS_T * _CH)
        sgi = jnp.stack([src, src + n_src_tab]).reshape(2, _NS_T, nch, _CH)
        dcl = jnp.minimum(dst, n_dst_tab - 1)
        dgi = jnp.stack([dcl, dcl + n_dst_tab]).reshape(2, _NS_T, nch, _CH)
        dsi = dst.reshape(_NS_T, nch, _CH)
        dsi_h = dst.reshape(2, _NS_T, nch // 2, _CH)
        return (sgi, dgi, dsi), dsi_h, e_feat

    g2m_idx, g2m_dsih, e_g2m_p = prep_edges(g2m_src, g2m_dst, e_g2m, N_MESH,
                                            N_GP, N_MESH)
    mm_idx, mm_dsih, e_mm_p = prep_edges(mesh_ei[0], mesh_ei[1], e_mm, N_MESH,
                                         N_MESH, N_MESH)
    m2g_idx, m2g_dsih, e_m2g_p = prep_edges(m2g_src, m2g_dst, e_m2g, N_GRID,
                                            N_MESH, N_GP)

    N_ACC_M = N_MESH + 128
    N_ACC_G = N_GP
    zeros = jnp.zeros((N_GP // _NS_T, 128), f32)
    aug = jnp.zeros((_CH, 128), f32).at[:, 0].set(1.0)
    cnt_g2m, cnt_mm, cnt_m2g = _counts_sc(
        [g2m_dsih, mm_dsih, m2g_dsih], aug, zeros,
        [N_ACC_M, N_ACC_M, N_ACC_G], [N_MESH, N_MESH, N_GP])

    h = _gru_call(Xt, w_ihT, w_hhT, p['gru_b_ih'], p['gru_b_hh'])
    xgc = _bn_concat_call(h, X_spa, p['bn_g'], p['bn_b'], p['spa_g'], p['spa_b'])

    (gw1, gb1), (gw2, gb2) = p['grid_enc']
    grid = _mlp2_call(xgc, jnp.pad(gw1, ((0, 0), (0, 0))), gb1, gw2, gb2,
                      ln=p['grid_ln'])
    (mw1, mb1), (mw2, mb2) = p['mesh_enc']
    mesh = _mlp2_call(mesh_feat.astype(f32), mw1, mb1, mw2, mb2,
                      ln=p['mesh_ln'])

    eg2m_p = _mlp2_call(e_g2m_p, eg_w1, eg_b1, eg_w2f, eg_b2f, split_out=True)
    emm_p = _mlp2_call(e_mm_p, em_w1, em_b1, em_w2f, em_b2f, split_out=True)
    em2g_p = _mlp2_call(e_m2g_p, e2_w1, e2_b1, e2_w2f, e2_b2f, split_out=True)

    gsrc_tab = _matmul_split_call(grid, g2m_w1s).reshape(2 * N_GP, 128)
    mdst_tab = _matmul_split_call(mesh, g2m_w1d).reshape(2 * N_MESH, 128)
    acc = _edge_stage_sc(gsrc_tab, mdst_tab, eg2m_p, *g2m_idx, zeros,
                         N_ACC_M, N_MESH)
    mesh = _post_stage_call(acc, cnt_g2m, mesh, g2m_w2, g2m_b2)

    for l in range(N_PROC):
        sd = _matmul_split_call(
            mesh, jnp.concatenate([proc_w1s[l], proc_w1d[l]], axis=1))
        s_tab = sd[0:2].reshape(2 * N_MESH, 128)
        d_tab = sd[2:4].reshape(2 * N_MESH, 128)
        acc = _edge_stage_sc(s_tab, d_tab, emm_p[2 * l:2 * l + 2], *mm_idx,
                             zeros, N_ACC_M, N_MESH)
        mesh = _post_stage_call(acc, cnt_mm, mesh, proc_w2[l], proc_b2[l],
                                ln=p['proc_ln'][l])

    msrc_tab = _matmul_split_call(mesh, m2g_w1s).reshape(2 * N_MESH, 128)
    gdst_tab = _matmul_split_call(grid, m2g_w1d).reshape(2 * N_GP, 128)
    acc = _edge_stage_sc(msrc_tab, gdst_tab, em2g_p, *m2g_idx, zeros,
                         N_ACC_G, N_GP)
    grid = _post_stage_call(acc, cnt_m2g, grid, m2g_w2, m2g_b2)

    output, logits, hidden = _decoder_call(grid, p)
    return (output[0:N_GRID, 0:4], logits[0:N_GRID, 0:4], hidden[0:N_GRID])


# speedup vs baseline: 4.1990x; 1.0824x over previous
import functools

import jax
import jax.numpy as jnp
from jax import lax
from jax.experimental import pallas as pl
from jax.experimental.pallas import tpu as pltpu
from jax.experimental.pallas import tpu_sc as plsc

N_GRID = 10000
N_GP = 10240
N_MESH = 2048
T = 8
GRU_H = 128
HID = 256
OUT_GRID = 64
N_PROC = 4

BN_EPS = 1e-5
LN_EPS = 1e-5


def _gru_call(xg, w_ihT, w_hhT, b_ih, b_hh, bn=1024):
    n = xg.shape[0]

    def body(x_ref, wi_ref, wh_ref, bi_ref, bh_ref, o_ref):
        h = jnp.zeros((bn, GRU_H), jnp.float32)
        wi = wi_ref[...]
        wh = wh_ref[...]
        bi = bi_ref[...]
        bh = bh_ref[...]
        for t in range(T):
            xt = x_ref[:, t * 10:(t + 1) * 10]
            gi = jnp.dot(xt, wi, preferred_element_type=jnp.float32) + bi
            gh = jnp.dot(h, wh, preferred_element_type=jnp.float32) + bh
            r = jax.nn.sigmoid(gi[:, 0:128] + gh[:, 0:128])
            z = jax.nn.sigmoid(gi[:, 128:256] + gh[:, 128:256])
            nn_ = jnp.tanh(gi[:, 256:384] + r * gh[:, 256:384])
            h = (1.0 - z) * nn_ + z * h
        o_ref[...] = h

    return pl.pallas_call(
        body,
        grid=(n // bn,),
        in_specs=[
            pl.BlockSpec((bn, T * 10), lambda i: (i, 0)),
            pl.BlockSpec((10, 384), lambda i: (0, 0)),
            pl.BlockSpec((GRU_H, 384), lambda i: (0, 0)),
            pl.BlockSpec((384,), lambda i: (0,)),
            pl.BlockSpec((384,), lambda i: (0,)),
        ],
        out_specs=pl.BlockSpec((bn, GRU_H), lambda i: (i, 0)),
        out_shape=jax.ShapeDtypeStruct((n, GRU_H), jnp.float32),
    )(xg, w_ihT, w_hhT, b_ih, b_hh)


def _bn_concat_call(h, x_spa, bn_g, bn_b, spa_g, spa_b):
    def body(h_ref, s_ref, g_ref, b_ref, sg_ref, sb_ref, o_ref):
        rows = lax.broadcasted_iota(jnp.int32, (N_GP, 1), 0)
        mask = (rows < N_GRID).astype(jnp.float32)
        inv_n = 1.0 / N_GRID
        h = h_ref[...]
        hm = h * mask
        mu = jnp.sum(hm, axis=0, keepdims=True) * inv_n
        d = (h - mu) * mask
        var = jnp.sum(d * d, axis=0, keepdims=True) * inv_n
        hn = (h - mu) * jax.lax.rsqrt(var + BN_EPS) * g_ref[...] + b_ref[...]
        s = s_ref[...]
        sm = s * mask
        smu = jnp.sum(sm, axis=0, keepdims=True) * inv_n
        sd = (s - smu) * mask
        svar = jnp.sum(sd * sd, axis=0, keepdims=True) * inv_n
        sn = (s - smu) * jax.lax.rsqrt(svar + BN_EPS) * sg_ref[...] + sb_ref[...]
        o_ref[:, 0:GRU_H] = hn
        o_ref[:, GRU_H:GRU_H + 4] = sn

    return pl.pallas_call(
        body,
        out_shape=jax.ShapeDtypeStruct((N_GP, GRU_H + 4), jnp.float32),
    )(h, x_spa, bn_g.reshape(1, -1), bn_b.reshape(1, -1),
      spa_g.reshape(1, -1), spa_b.reshape(1, -1))


def _mlp2_call(x, w1, b1, w2, b2, ln=None, split_out=False, bn=None):
    n, k = x.shape
    dh = w1.shape[1]
    dout = w2.shape[1]
    if bn is None:
        bn = n if n <= 4096 else 1024
    assert n % bn == 0
    nsplit = dout // 128
    has_ln = ln is not None

    def body(x_ref, w1_ref, b1_ref, w2_ref, b2_ref, *rest):
        if has_ln:
            g_ref, be_ref, o_ref = rest
        else:
            (o_ref,) = rest
        h = jnp.maximum(
            jnp.dot(x_ref[...], w1_ref[...],
                    preferred_element_type=jnp.float32) + b1_ref[...], 0.0)
        y = jnp.dot(h, w2_ref[...], preferred_element_type=jnp.float32) + b2_ref[...]
        if has_ln:
            mu = jnp.mean(y, axis=-1, keepdims=True)
            d = y - mu
            var = jnp.mean(d * d, axis=-1, keepdims=True)
            y = d * jax.lax.rsqrt(var + LN_EPS) * g_ref[...] + be_ref[...]
        if split_out:
            for j in range(nsplit):
                o_ref[j, :, :] = y[:, 128 * j:128 * (j + 1)]
        else:
            o_ref[...] = y

    in_specs = [
        pl.BlockSpec((bn, k), lambda i: (i, 0)),
        pl.BlockSpec((k, dh), lambda i: (0, 0)),
        pl.BlockSpec((1, dh), lambda i: (0, 0)),
        pl.BlockSpec((dh, dout), lambda i: (0, 0)),
        pl.BlockSpec((1, dout), lambda i: (0, 0)),
    ]
    args = [x, w1, b1.reshape(1, -1), w2, b2.reshape(1, -1)]
    if has_ln:
        in_specs += [pl.BlockSpec((1, dout), lambda i: (0, 0)),
                     pl.BlockSpec((1, dout), lambda i: (0, 0))]
        args += [ln[0].reshape(1, -1), ln[1].reshape(1, -1)]
    if split_out:
        out_specs = pl.BlockSpec((nsplit, bn, 128), lambda i: (0, i, 0))
        out_shape = jax.ShapeDtypeStruct((nsplit, n, 128), jnp.float32)
    else:
        out_specs = pl.BlockSpec((bn, dout), lambda i: (i, 0))
        out_shape = jax.ShapeDtypeStruct((n, dout), jnp.float32)
    return pl.pallas_call(
        body, grid=(n // bn,), in_specs=in_specs,
        out_specs=out_specs, out_shape=out_shape)(*args)


def _matmul_split_call(x, w, bn=None):
    n, k = x.shape
    dout = w.shape[1]
    nsplit = dout // 128
    if bn is None:
        bn = n if n <= 4096 else 1024
    assert n % bn == 0

    def body(x_ref, w_ref, o_ref):
        y = jnp.dot(x_ref[...], w_ref[...], preferred_element_type=jnp.float32)
        for j in range(nsplit):
            o_ref[j, :, :] = y[:, 128 * j:128 * (j + 1)]

    return pl.pallas_call(
        body, grid=(n // bn,),
        in_specs=[pl.BlockSpec((bn, k), lambda i: (i, 0)),
                  pl.BlockSpec((k, dout), lambda i: (0, 0))],
        out_specs=pl.BlockSpec((nsplit, bn, 128), lambda i: (0, i, 0)),
        out_shape=jax.ShapeDtypeStruct((nsplit, n, 128), jnp.float32))(x, w)


def _post_stage_call(acc, cnt, prev, w2, b2, ln=None, bn=None):
    n = prev.shape[0]
    if bn is None:
        bn = n if n <= 4096 else 1024
    assert n % bn == 0
    has_ln = ln is not None

    def body(a_ref, c_ref, p_ref, w_ref, b_ref, *rest):
        if has_ln:
            g_ref, be_ref, o_ref = rest
        else:
            (o_ref,) = rest
        a0 = a_ref[0, :, :]
        a1 = a_ref[1, :, :]
        cnt_col = c_ref[0, :, 0:1] + c_ref[1, :, 0:1]
        y = (p_ref[...]
             + jnp.dot(a0, w_ref[0:128, :], preferred_element_type=jnp.float32)
             + jnp.dot(a1, w_ref[128:256, :], preferred_element_type=jnp.float32)
             + cnt_col * b_ref[...])
        if has_ln:
            mu = jnp.mean(y, axis=-1, keepdims=True)
            d = y - mu
            var = jnp.mean(d * d, axis=-1, keepdims=True)
            y = d * jax.lax.rsqrt(var + LN_EPS) * g_ref[...] + be_ref[...]
        o_ref[...] = y

    in_specs = [
        pl.BlockSpec((2, bn, 128), lambda i: (0, i, 0)),
        pl.BlockSpec((2, bn, 128), lambda i: (0, i, 0)),
        pl.BlockSpec((bn, HID), lambda i: (i, 0)),
        pl.BlockSpec((HID, HID), lambda i: (0, 0)),
        pl.BlockSpec((1, HID), lambda i: (0, 0)),
    ]
    args = [acc, cnt, prev, w2, b2.reshape(1, -1)]
    if has_ln:
        in_specs += [pl.BlockSpec((1, HID), lambda i: (0, 0)),
                     pl.BlockSpec((1, HID), lambda i: (0, 0))]
        args += [ln[0].reshape(1, -1), ln[1].reshape(1, -1)]
    return pl.pallas_call(
        body, grid=(n // bn,), in_specs=in_specs,
        out_specs=pl.BlockSpec((bn, HID), lambda i: (i, 0)),
        out_shape=jax.ShapeDtypeStruct((n, HID), jnp.float32))(*args)


def _decoder_call(x, p, bn=1024):
    n = x.shape[0]
    (d1w, d1b), (d2w, d2b) = p['dec']
    l1w, l1b = p['lin1']
    l2w, l2b = p['lin2']
    ow, ob = p['out']

    def body(x_ref, d1w_r, d1b_r, d2w_r, d2b_r, l1w_r, l1b_r, l2w_r, l2b_r,
             ow_r, ob_r, out_r, log_r, hid_r):
        h = jnp.maximum(jnp.dot(x_ref[...], d1w_r[...],
                                preferred_element_type=jnp.float32) + d1b_r[...], 0.0)
        y = jnp.dot(h, d2w_r[...], preferred_element_type=jnp.float32) + d2b_r[...]
        y = jnp.maximum(jnp.dot(y, l1w_r[...],
                                preferred_element_type=jnp.float32) + l1b_r[...], 0.0)
        hid = jnp.maximum(jnp.dot(y, l2w_r[...],
                                  preferred_element_type=jnp.float32) + l2b_r[...], 0.0)
        logits = jnp.dot(hid, ow_r[...], preferred_element_type=jnp.float32) + ob_r[...]
        cmask = (lax.broadcasted_iota(jnp.int32, (1, 8), 1) < 4).astype(jnp.float32)
        m = jnp.max(logits - 1e30 * (1.0 - cmask), axis=-1, keepdims=True)
        e = jnp.exp(logits - m) * cmask
        out_r[...] = e / jnp.sum(e, axis=-1, keepdims=True)
        log_r[...] = logits
        hid_r[...] = hid

    wspec = lambda shape: pl.BlockSpec(shape, lambda i: tuple(0 for _ in shape))
    return pl.pallas_call(
        body, grid=(n // bn,),
        in_specs=[
            pl.BlockSpec((bn, HID), lambda i: (i, 0)),
            wspec((HID, HID)), wspec((1, HID)),
            wspec((HID, OUT_GRID)), wspec((1, OUT_GRID)),
            wspec((OUT_GRID, 64)), wspec((1, 64)),
            wspec((64, 64)), wspec((1, 64)),
            wspec((64, 8)), wspec((1, 8)),
        ],
        out_specs=[
            pl.BlockSpec((bn, 8), lambda i: (i, 0)),
            pl.BlockSpec((bn, 8), lambda i: (i, 0)),
            pl.BlockSpec((bn, 64), lambda i: (i, 0)),
        ],
        out_shape=[
            jax.ShapeDtypeStruct((n, 8), jnp.float32),
            jax.ShapeDtypeStruct((n, 8), jnp.float32),
            jax.ShapeDtypeStruct((n, 64), jnp.float32),
        ],
    )(x, d1w, d1b.reshape(1, -1), d2w, d2b.reshape(1, -1),
      l1w, l1b.reshape(1, -1), l2w, l2b.reshape(1, -1),
      jnp.pad(ow, ((0, 0), (0, 4))), jnp.pad(ob, (0, 4)).reshape(1, -1))


_NS_T = 16
_CH = 64


def _edge_stage_sc(src_tab, dst_tab, e_p, sgi, dgi, dsi, zeros, n_acc_sp,
                   n_out, dbuf=True):
    e_pad = e_p.shape[1]
    e_per_tile = e_pad // _NS_T
    n_chunks = e_per_tile // _CH
    rpt_z = n_acc_sp // _NS_T
    rpt_o = n_out // _NS_T
    mesh = plsc.VectorSubcoreMesh(core_axis_name="c", subcore_axis_name="s")

    assert n_chunks % 2 == 0
    nb = 2 if dbuf else 1

    @functools.partial(
        pl.kernel,
        out_type=jax.ShapeDtypeStruct((2, n_out, 128), jnp.float32),
        mesh=mesh,
        scratch_types=[
            pltpu.VMEM((n_chunks, _CH), jnp.int32),
            pltpu.VMEM((n_chunks, _CH), jnp.int32),
            pltpu.VMEM((n_chunks, _CH), jnp.int32),
            pltpu.VMEM((nb, _CH, 128), jnp.float32),
            pltpu.VMEM((nb, _CH, 128), jnp.float32),
            pltpu.VMEM((nb, _CH, 128), jnp.float32),
            pltpu.VMEM((nb, _CH, 128), jnp.float32),
            pltpu.VMEM_SHARED((n_acc_sp, 128), jnp.float32),
            [pltpu.SemaphoreType.DMA] * 6,
        ],
    )
    def k(src_tab_h, dst_tab_h, e_p_h, sgi_h, dgi_h, dsi_h, zeros_h, out_h,
          sidx, didx, dsidx, srow, drow, erow, msg, acc, sems):
        c = lax.axis_index("c")
        s = lax.axis_index("s")
        base = s * e_per_tile

        pltpu.sync_copy(sgi_h.at[c, s], sidx)
        pltpu.sync_copy(dgi_h.at[c, s], didx)
        pltpu.sync_copy(dsi_h.at[s], dsidx)
        pltpu.sync_copy(zeros_h.at[pl.ds(0, rpt_z)],
                        acc.at[pl.ds(s * rpt_z, rpt_z)])
        plsc.subcore_barrier()

        def issue(j, b):
            pltpu.async_copy(src_tab_h.at[sidx.at[j]], srow.at[b], sems[b])
            pltpu.async_copy(dst_tab_h.at[didx.at[j]], drow.at[b], sems[2 + b])
            pltpu.async_copy(e_p_h.at[c, pl.ds(base + j * _CH, _CH)],
                             erow.at[b], sems[4 + b])

        def wait(j, b):
            pltpu.make_async_copy(src_tab_h.at[sidx.at[j]], srow.at[b],
                                  sems[b]).wait()
            pltpu.make_async_copy(dst_tab_h.at[didx.at[j]], drow.at[b],
                                  sems[2 + b]).wait()
            pltpu.make_async_copy(e_p_h.at[c, pl.ds(base + j * _CH, _CH)],
                                  erow.at[b], sems[4 + b]).wait()

        def compute(j, b):
            @plsc.parallel_loop(0, _CH, unroll=2)
            def row(i):
                for jj in range(8):
                    sl = pl.ds(16 * jj, 16)
                    msg[b, i, sl] = jnp.maximum(
                        srow[b, i, sl] + drow[b, i, sl] + erow[b, i, sl], 0.0)

            pltpu.sync_copy(msg.at[b], acc.at[dsidx.at[j]], add=True)

        if dbuf:
            issue(0, 0)

            def pair(t, _):
                j0 = 2 * t
                j1 = j0 + 1
                issue(j1, 1)
                wait(j0, 0)
                compute(j0, 0)
                issue(jnp.minimum(j0 + 2, n_chunks - 1), 0)
                wait(j1, 1)
                compute(j1, 1)
                return _

            lax.fori_loop(0, n_chunks // 2, pair, None)
            wait(n_chunks - 1, 0)
        else:
            def chunk(j, _):
                issue(j, 0)
                wait(j, 0)
                compute(j, 0)
                return _

            lax.fori_loop(0, n_chunks, chunk, None)
        plsc.subcore_barrier()
        pltpu.sync_copy(acc.at[pl.ds(s * rpt_o, rpt_o)],
                        out_h.at[c, pl.ds(s * rpt_o, rpt_o)])

    return k(src_tab, dst_tab, e_p, sgi, dgi, dsi, zeros)


def _counts_sc(dsi_list, aug, zeros, n_accs, n_outs):
    n_sp = max(n_accs)
    mesh = plsc.VectorSubcoreMesh(core_axis_name="c", subcore_axis_name="s")
    max_nch = max(d.shape[2] for d in dsi_list)

    @functools.partial(
        pl.kernel,
        out_type=tuple(jax.ShapeDtypeStruct((2, n, 128), jnp.float32)
                       for n in n_outs),
        mesh=mesh,
        scratch_types=[
            pltpu.VMEM((max_nch, _CH), jnp.int32),
            pltpu.VMEM((_CH, 128), jnp.float32),
            pltpu.VMEM_SHARED((n_sp, 128), jnp.float32),
        ],
    )
    def k(*refs):
        nl = len(dsi_list)
        dsis = refs[0:nl]
        aug_h, zeros_h = refs[nl], refs[nl + 1]
        outs = refs[nl + 2:2 * nl + 2]
        dsidx, augv, acc = refs[2 * nl + 2:]
        c = lax.axis_index("c")
        s = lax.axis_index("s")
        pltpu.sync_copy(aug_h, augv)
        for l in range(nl):
            n_acc, n_out = n_accs[l], n_outs[l]
            nch = dsi_list[l].shape[2]
            rpt_z = n_acc // _NS_T
            rpt_o = n_out // _NS_T
            pltpu.sync_copy(zeros_h.at[pl.ds(0, rpt_z)],
                            acc.at[pl.ds(s * rpt_z, rpt_z)])
            plsc.subcore_barrier()
            pltpu.sync_copy(dsis[l].at[c, s], dsidx.at[pl.ds(0, nch)])

            def chunk(j, _):
                pltpu.sync_copy(augv, acc.at[dsidx.at[j]], add=True)
                return _

            lax.fori_loop(0, nch, chunk, None)
            plsc.subcore_barrier()
            pltpu.sync_copy(acc.at[pl.ds(s * rpt_o, rpt_o)],
                            outs[l].at[c, pl.ds(s * rpt_o, rpt_o)])
            plsc.subcore_barrier()

    return k(*dsi_list, aug, zeros)


def kernel(X, mesh_feat, mesh_ei, g2m_src, g2m_dst, m2g_src, m2g_dst,
           e_mm, e_g2m, e_m2g, params):
    p = params
    f32 = jnp.float32

    Xt = jnp.transpose(X[:, 0:10, :], (0, 2, 1)).reshape(N_GRID, T * 10)
    Xt = jnp.pad(Xt, ((0, N_GP - N_GRID), (0, 0)))
    X_spa = jnp.pad(X[:, 10:14, -1], ((0, N_GP - N_GRID), (0, 0)))

    w_ihT = p['gru_W_ih'].T.astype(f32)
    w_hhT = p['gru_W_hh'].T.astype(f32)

    def msg_split(w):
        return w[0:HID], w[HID:2 * HID], w[2 * HID:3 * HID]

    (g2m_w1, g2m_b1), (g2m_w2, g2m_b2) = p['g2m_msg']
    g2m_w1s, g2m_w1d, g2m_w1e = msg_split(g2m_w1)
    (m2g_w1, m2g_b1), (m2g_w2, m2g_b2) = p['m2g_msg']
    m2g_w1s, m2g_w1d, m2g_w1e = msg_split(m2g_w1)
    proc_w1s, proc_w1d, proc_w1e, proc_w2, proc_b2, proc_b1 = [], [], [], [], [], []
    for l in range(N_PROC):
        (w1, b1), (w2, b2) = p['proc'][l]
        s, d, e = msg_split(w1)
        proc_w1s.append(s); proc_w1d.append(d); proc_w1e.append(e)
        proc_w2.append(w2); proc_b2.append(b2); proc_b1.append(b1)

    (eg_w1, eg_b1), (eg_w2, eg_b2) = p['eg2m_enc']
    eg_w2f = eg_w2 @ g2m_w1e
    eg_b2f = eg_b2 @ g2m_w1e + g2m_b1
    (em_w1, em_b1), (em_w2, em_b2) = p['emm_enc']
    em_w2f = jnp.concatenate([em_w2 @ proc_w1e[l] for l in range(N_PROC)], axis=1)
    em_b2f = jnp.concatenate([em_b2 @ proc_w1e[l] + proc_b1[l]
                              for l in range(N_PROC)])
    (e2_w1, e2_b1), (e2_w2, e2_b2) = p['em2g_enc']
    e2_w2f = e2_w2 @ m2g_w1e
    e2_b2f = e2_b2 @ m2g_w1e + m2g_b1

    def prep_edges(src, dst, e_feat, dummy_dst, n_src_tab, n_dst_tab):
        E = src.shape[0]
        Ep = ((E + 2047) // 2048) * 2048
        src = jnp.pad(src.astype(jnp.int32), (0, Ep - E))
        dst = jnp.pad(dst.astype(jnp.int32), (0, Ep - E),
                      constant_values=dummy_dst)
        e_feat = jnp.pad(e_feat.astype(f32), ((0, Ep - E), (0, 0)))
        nch = Ep // (_NS_T * _CH)
        sgi = jnp.stack([src, src + n_src_tab]).reshape(2, _NS_T, nch, _CH)
        dcl = jnp.minimum(dst, n_dst_tab - 1)
        dgi = jnp.stack([dcl, dcl + n_dst_tab]).reshape(2, _NS_T, nch, _CH)
        dsi = dst.reshape(_NS_T, nch, _CH)
        dsi_h = dst.reshape(2, _NS_T, nch // 2, _CH)
        return (sgi, dgi, dsi), dsi_h, e_feat

    g2m_idx, g2m_dsih, e_g2m_p = prep_edges(g2m_src, g2m_dst, e_g2m, N_MESH,
                                            N_GP, N_MESH)
    mm_idx, mm_dsih, e_mm_p = prep_edges(mesh_ei[0], mesh_ei[1], e_mm, N_MESH,
                                         N_MESH, N_MESH)
    m2g_idx, m2g_dsih, e_m2g_p = prep_edges(m2g_src, m2g_dst, e_m2g, N_GRID,
                                            N_MESH, N_GP)

    N_ACC_M = N_MESH + 128
    N_ACC_G = N_GP
    zeros = jnp.zeros((N_GP // _NS_T, 128), f32)
    aug = jnp.zeros((_CH, 128), f32).at[:, 0].set(1.0)
    cnt_g2m, cnt_mm, cnt_m2g = _counts_sc(
        [g2m_dsih, mm_dsih, m2g_dsih], aug, zeros,
        [N_ACC_M, N_ACC_M, N_ACC_G], [N_MESH, N_MESH, N_GP])

    h = _gru_call(Xt, w_ihT, w_hhT, p['gru_b_ih'], p['gru_b_hh'])
    xgc = _bn_concat_call(h, X_spa, p['bn_g'], p['bn_b'], p['spa_g'], p['spa_b'])

    (gw1, gb1), (gw2, gb2) = p['grid_enc']
    grid = _mlp2_call(xgc, jnp.pad(gw1, ((0, 0), (0, 0))), gb1, gw2, gb2,
                      ln=p['grid_ln'])
    (mw1, mb1), (mw2, mb2) = p['mesh_enc']
    mesh = _mlp2_call(mesh_feat.astype(f32), mw1, mb1, mw2, mb2,
                      ln=p['mesh_ln'])

    eg2m_p = _mlp2_call(e_g2m_p, eg_w1, eg_b1, eg_w2f, eg_b2f, split_out=True)
    emm_p = _mlp2_call(e_mm_p, em_w1, em_b1, em_w2f, em_b2f, split_out=True)
    em2g_p = _mlp2_call(e_m2g_p, e2_w1, e2_b1, e2_w2f, e2_b2f, split_out=True)

    gsrc_tab = _matmul_split_call(grid, g2m_w1s).reshape(2 * N_GP, 128)
    mdst_tab = _matmul_split_call(mesh, g2m_w1d).reshape(2 * N_MESH, 128)
    acc = _edge_stage_sc(gsrc_tab, mdst_tab, eg2m_p, *g2m_idx, zeros,
                         N_ACC_M, N_MESH)
    mesh = _post_stage_call(acc, cnt_g2m, mesh, g2m_w2, g2m_b2)

    for l in range(N_PROC):
        sd = _matmul_split_call(
            mesh, jnp.concatenate([proc_w1s[l], proc_w1d[l]], axis=1))
        s_tab = sd[0:2].reshape(2 * N_MESH, 128)
        d_tab = sd[2:4].reshape(2 * N_MESH, 128)
        acc = _edge_stage_sc(s_tab, d_tab, emm_p[2 * l:2 * l + 2], *mm_idx,
                             zeros, N_ACC_M, N_MESH)
        mesh = _post_stage_call(acc, cnt_mm, mesh, proc_w2[l], proc_b2[l],
                                ln=p['proc_ln'][l])

    msrc_tab = _matmul_split_call(mesh, m2g_w1s).reshape(2 * N_MESH, 128)
    gdst_tab = _matmul_split_call(grid, m2g_w1d).reshape(2 * N_GP, 128)
    acc = _edge_stage_sc(msrc_tab, gdst_tab, em2g_p, *m2g_idx, zeros,
                         N_ACC_G, N_GP, dbuf=False)
    grid = _post_stage_call(acc, cnt_m2g, grid, m2g_w2, m2g_b2)

    output, logits, hidden = _decoder_call(grid, p)
    return (output[0:N_GRID, 0:4], logits[0:N_GRID, 0:4], hidden[0:N_GRID])


# speedup vs baseline: 4.2377x; 1.0092x over previous
import functools

import jax
import jax.numpy as jnp
from jax import lax
from jax.experimental import pallas as pl
from jax.experimental.pallas import tpu as pltpu
from jax.experimental.pallas import tpu_sc as plsc

N_GRID = 10000
N_GP = 10240
N_MESH = 2048
T = 8
GRU_H = 128
HID = 256
OUT_GRID = 64
N_PROC = 4

BN_EPS = 1e-5
LN_EPS = 1e-5


def _gru_call(xg, w_ihT, w_hhT, b_ih, b_hh, bn=1024):
    n = xg.shape[0]

    def body(x_ref, wi_ref, wh_ref, bi_ref, bh_ref, o_ref):
        h = jnp.zeros((bn, GRU_H), jnp.float32)
        wi = wi_ref[...]
        wh = wh_ref[...]
        bi = bi_ref[...]
        bh = bh_ref[...]
        for t in range(T):
            xt = x_ref[:, t * 10:(t + 1) * 10]
            gi = jnp.dot(xt, wi, preferred_element_type=jnp.float32) + bi
            gh = jnp.dot(h, wh, preferred_element_type=jnp.float32) + bh
            r = jax.nn.sigmoid(gi[:, 0:128] + gh[:, 0:128])
            z = jax.nn.sigmoid(gi[:, 128:256] + gh[:, 128:256])
            nn_ = jnp.tanh(gi[:, 256:384] + r * gh[:, 256:384])
            h = (1.0 - z) * nn_ + z * h
        o_ref[...] = h

    return pl.pallas_call(
        body,
        grid=(n // bn,),
        in_specs=[
            pl.BlockSpec((bn, T * 10), lambda i: (i, 0)),
            pl.BlockSpec((10, 384), lambda i: (0, 0)),
            pl.BlockSpec((GRU_H, 384), lambda i: (0, 0)),
            pl.BlockSpec((384,), lambda i: (0,)),
            pl.BlockSpec((384,), lambda i: (0,)),
        ],
        out_specs=pl.BlockSpec((bn, GRU_H), lambda i: (i, 0)),
        out_shape=jax.ShapeDtypeStruct((n, GRU_H), jnp.float32),
    )(xg, w_ihT, w_hhT, b_ih, b_hh)


def _bn_concat_call(h, x_spa, bn_g, bn_b, spa_g, spa_b):
    def body(h_ref, s_ref, g_ref, b_ref, sg_ref, sb_ref, o_ref):
        rows = lax.broadcasted_iota(jnp.int32, (N_GP, 1), 0)
        mask = (rows < N_GRID).astype(jnp.float32)
        inv_n = 1.0 / N_GRID
        h = h_ref[...]
        hm = h * mask
        mu = jnp.sum(hm, axis=0, keepdims=True) * inv_n
        d = (h - mu) * mask
        var = jnp.sum(d * d, axis=0, keepdims=True) * inv_n
        hn = (h - mu) * jax.lax.rsqrt(var + BN_EPS) * g_ref[...] + b_ref[...]
        s = s_ref[...]
        sm = s * mask
        smu = jnp.sum(sm, axis=0, keepdims=True) * inv_n
        sd = (s - smu) * mask
        svar = jnp.sum(sd * sd, axis=0, keepdims=True) * inv_n
        sn = (s - smu) * jax.lax.rsqrt(svar + BN_EPS) * sg_ref[...] + sb_ref[...]
        o_ref[:, 0:GRU_H] = hn
        o_ref[:, GRU_H:GRU_H + 4] = sn

    return pl.pallas_call(
        body,
        out_shape=jax.ShapeDtypeStruct((N_GP, GRU_H + 4), jnp.float32),
    )(h, x_spa, bn_g.reshape(1, -1), bn_b.reshape(1, -1),
      spa_g.reshape(1, -1), spa_b.reshape(1, -1))


def _mlp2_call(x, w1, b1, w2, b2, ln=None, split_out=False, bn=None,
               proj_w=None):
    n, k = x.shape
    dh = w1.shape[1]
    dout = w2.shape[1]
    if bn is None:
        bn = n if n <= 4096 else 1024
    assert n % bn == 0
    nsplit = dout // 128
    np_ = proj_w.shape[1] // 128 if proj_w is not None else 0
    has_ln = ln is not None

    def body(x_ref, w1_ref, b1_ref, w2_ref, b2_ref, *rest):
        rest = list(rest)
        if has_ln:
            g_ref, be_ref = rest.pop(0), rest.pop(0)
        if proj_w is not None:
            pw_ref = rest.pop(0)
        o_ref = rest.pop(0)
        h = jnp.maximum(
            jnp.dot(x_ref[...], w1_ref[...],
                    preferred_element_type=jnp.float32) + b1_ref[...], 0.0)
        y = jnp.dot(h, w2_ref[...], preferred_element_type=jnp.float32) + b2_ref[...]
        if has_ln:
            mu = jnp.mean(y, axis=-1, keepdims=True)
            d = y - mu
            var = jnp.mean(d * d, axis=-1, keepdims=True)
            y = d * jax.lax.rsqrt(var + LN_EPS) * g_ref[...] + be_ref[...]
        if split_out:
            for j in range(nsplit):
                o_ref[j, :, :] = y[:, 128 * j:128 * (j + 1)]
        else:
            o_ref[...] = y
        if proj_w is not None:
            op_ref = rest.pop(0)
            z = jnp.dot(y, pw_ref[...], preferred_element_type=jnp.float32)
            for j in range(np_):
                op_ref[j, :, :] = z[:, 128 * j:128 * (j + 1)]

    in_specs = [
        pl.BlockSpec((bn, k), lambda i: (i, 0)),
        pl.BlockSpec((k, dh), lambda i: (0, 0)),
        pl.BlockSpec((1, dh), lambda i: (0, 0)),
        pl.BlockSpec((dh, dout), lambda i: (0, 0)),
        pl.BlockSpec((1, dout), lambda i: (0, 0)),
    ]
    args = [x, w1, b1.reshape(1, -1), w2, b2.reshape(1, -1)]
    if has_ln:
        in_specs += [pl.BlockSpec((1, dout), lambda i: (0, 0)),
                     pl.BlockSpec((1, dout), lambda i: (0, 0))]
        args += [ln[0].reshape(1, -1), ln[1].reshape(1, -1)]
    if proj_w is not None:
        in_specs.append(pl.BlockSpec((dout, np_ * 128), lambda i: (0, 0)))
        args.append(proj_w)
    if split_out:
        out_specs = pl.BlockSpec((nsplit, bn, 128), lambda i: (0, i, 0))
        out_shape = jax.ShapeDtypeStruct((nsplit, n, 128), jnp.float32)
    else:
        out_specs = pl.BlockSpec((bn, dout), lambda i: (i, 0))
        out_shape = jax.ShapeDtypeStruct((n, dout), jnp.float32)
    if proj_w is not None:
        out_specs = [out_specs,
                     pl.BlockSpec((np_, bn, 128), lambda i: (0, i, 0))]
        out_shape = [out_shape,
                     jax.ShapeDtypeStruct((np_, n, 128), jnp.float32)]
    return pl.pallas_call(
        body, grid=(n // bn,), in_specs=in_specs,
        out_specs=out_specs, out_shape=out_shape)(*args)


def _matmul_split_call(x, w, bn=None):
    n, k = x.shape
    dout = w.shape[1]
    nsplit = dout // 128
    if bn is None:
        bn = n if n <= 4096 else 1024
    assert n % bn == 0

    def body(x_ref, w_ref, o_ref):
        y = jnp.dot(x_ref[...], w_ref[...], preferred_element_type=jnp.float32)
        for j in range(nsplit):
            o_ref[j, :, :] = y[:, 128 * j:128 * (j + 1)]

    return pl.pallas_call(
        body, grid=(n // bn,),
        in_specs=[pl.BlockSpec((bn, k), lambda i: (i, 0)),
                  pl.BlockSpec((k, dout), lambda i: (0, 0))],
        out_specs=pl.BlockSpec((nsplit, bn, 128), lambda i: (0, i, 0)),
        out_shape=jax.ShapeDtypeStruct((nsplit, n, 128), jnp.float32))(x, w)


def _post_stage_call(acc, cnt, prev, w2, b2, ln=None, bn=None, proj_w=None):
    n = prev.shape[0]
    if bn is None:
        bn = n if n <= 4096 else 1024
    assert n % bn == 0
    has_ln = ln is not None
    np_ = proj_w.shape[1] // 128 if proj_w is not None else 0

    def body(a_ref, c_ref, p_ref, w_ref, b_ref, *rest):
        rest = list(rest)
        if has_ln:
            g_ref, be_ref = rest.pop(0), rest.pop(0)
        if proj_w is not None:
            pw_ref = rest.pop(0)
        o_ref = rest.pop(0)
        a0 = a_ref[0, :, :]
        a1 = a_ref[1, :, :]
        cnt_col = c_ref[0, :, 0:1] + c_ref[1, :, 0:1]
        y = (p_ref[...]
             + jnp.dot(a0, w_ref[0:128, :], preferred_element_type=jnp.float32)
             + jnp.dot(a1, w_ref[128:256, :], preferred_element_type=jnp.float32)
             + cnt_col * b_ref[...])
        if has_ln:
            mu = jnp.mean(y, axis=-1, keepdims=True)
            d = y - mu
            var = jnp.mean(d * d, axis=-1, keepdims=True)
            y = d * jax.lax.rsqrt(var + LN_EPS) * g_ref[...] + be_ref[...]
        o_ref[...] = y
        if proj_w is not None:
            op_ref = rest.pop(0)
            z = jnp.dot(y, pw_ref[...], preferred_element_type=jnp.float32)
            for j in range(np_):
                op_ref[j, :, :] = z[:, 128 * j:128 * (j + 1)]

    in_specs = [
        pl.BlockSpec((2, bn, 128), lambda i: (0, i, 0)),
        pl.BlockSpec((2, bn, 128), lambda i: (0, i, 0)),
        pl.BlockSpec((bn, HID), lambda i: (i, 0)),
        pl.BlockSpec((HID, HID), lambda i: (0, 0)),
        pl.BlockSpec((1, HID), lambda i: (0, 0)),
    ]
    args = [acc, cnt, prev, w2, b2.reshape(1, -1)]
    if has_ln:
        in_specs += [pl.BlockSpec((1, HID), lambda i: (0, 0)),
                     pl.BlockSpec((1, HID), lambda i: (0, 0))]
        args += [ln[0].reshape(1, -1), ln[1].reshape(1, -1)]
    if proj_w is not None:
        in_specs.append(pl.BlockSpec((HID, np_ * 128), lambda i: (0, 0)))
        args.append(proj_w)
    out_specs = pl.BlockSpec((bn, HID), lambda i: (i, 0))
    out_shape = jax.ShapeDtypeStruct((n, HID), jnp.float32)
    if proj_w is not None:
        out_specs = [out_specs,
                     pl.BlockSpec((np_, bn, 128), lambda i: (0, i, 0))]
        out_shape = [out_shape,
                     jax.ShapeDtypeStruct((np_, n, 128), jnp.float32)]
    return pl.pallas_call(
        body, grid=(n // bn,), in_specs=in_specs,
        out_specs=out_specs, out_shape=out_shape)(*args)


def _decoder_call(x, p, bn=1024):
    n = x.shape[0]
    (d1w, d1b), (d2w, d2b) = p['dec']
    l1w, l1b = p['lin1']
    l2w, l2b = p['lin2']
    ow, ob = p['out']

    def body(x_ref, d1w_r, d1b_r, d2w_r, d2b_r, l1w_r, l1b_r, l2w_r, l2b_r,
             ow_r, ob_r, out_r, log_r, hid_r):
        h = jnp.maximum(jnp.dot(x_ref[...], d1w_r[...],
                                preferred_element_type=jnp.float32) + d1b_r[...], 0.0)
        y = jnp.dot(h, d2w_r[...], preferred_element_type=jnp.float32) + d2b_r[...]
        y = jnp.maximum(jnp.dot(y, l1w_r[...],
                                preferred_element_type=jnp.float32) + l1b_r[...], 0.0)
        hid = jnp.maximum(jnp.dot(y, l2w_r[...],
                                  preferred_element_type=jnp.float32) + l2b_r[...], 0.0)
        logits = jnp.dot(hid, ow_r[...], preferred_element_type=jnp.float32) + ob_r[...]
        cmask = (lax.broadcasted_iota(jnp.int32, (1, 8), 1) < 4).astype(jnp.float32)
        m = jnp.max(logits - 1e30 * (1.0 - cmask), axis=-1, keepdims=True)
        e = jnp.exp(logits - m) * cmask
        out_r[...] = e / jnp.sum(e, axis=-1, keepdims=True)
        log_r[...] = logits
        hid_r[...] = hid

    wspec = lambda shape: pl.BlockSpec(shape, lambda i: tuple(0 for _ in shape))
    return pl.pallas_call(
        body, grid=(n // bn,),
        in_specs=[
            pl.BlockSpec((bn, HID), lambda i: (i, 0)),
            wspec((HID, HID)), wspec((1, HID)),
            wspec((HID, OUT_GRID)), wspec((1, OUT_GRID)),
            wspec((OUT_GRID, 64)), wspec((1, 64)),
            wspec((64, 64)), wspec((1, 64)),
            wspec((64, 8)), wspec((1, 8)),
        ],
        out_specs=[
            pl.BlockSpec((bn, 8), lambda i: (i, 0)),
            pl.BlockSpec((bn, 8), lambda i: (i, 0)),
            pl.BlockSpec((bn, 64), lambda i: (i, 0)),
        ],
        out_shape=[
            jax.ShapeDtypeStruct((n, 8), jnp.float32),
            jax.ShapeDtypeStruct((n, 8), jnp.float32),
            jax.ShapeDtypeStruct((n, 64), jnp.float32),
        ],
    )(x, d1w, d1b.reshape(1, -1), d2w, d2b.reshape(1, -1),
      l1w, l1b.reshape(1, -1), l2w, l2b.reshape(1, -1),
      jnp.pad(ow, ((0, 0), (0, 4))), jnp.pad(ob, (0, 4)).reshape(1, -1))


_NS_T = 16
_CH = 64


def _edge_stage_sc(src_tab, dst_tab, e_p, sgi, dgi, dsi, zeros, n_acc_sp,
                   n_out, dbuf=True):
    e_pad = e_p.shape[1]
    e_per_tile = e_pad // _NS_T
    n_chunks = e_per_tile // _CH
    rpt_z = n_acc_sp // _NS_T
    rpt_o = n_out // _NS_T
    mesh = plsc.VectorSubcoreMesh(core_axis_name="c", subcore_axis_name="s")

    assert n_chunks % 2 == 0
    nb = 2 if dbuf else 1

    @functools.partial(
        pl.kernel,
        out_type=jax.ShapeDtypeStruct((2, n_out, 128), jnp.float32),
        mesh=mesh,
        scratch_types=[
            pltpu.VMEM((n_chunks, _CH), jnp.int32),
            pltpu.VMEM((n_chunks, _CH), jnp.int32),
            pltpu.VMEM((n_chunks, _CH), jnp.int32),
            pltpu.VMEM((nb, _CH, 128), jnp.float32),
            pltpu.VMEM((nb, _CH, 128), jnp.float32),
            pltpu.VMEM((nb, _CH, 128), jnp.float32),
            pltpu.VMEM((nb, _CH, 128), jnp.float32),
            pltpu.VMEM_SHARED((n_acc_sp, 128), jnp.float32),
            [pltpu.SemaphoreType.DMA] * 6,
        ],
    )
    def k(src_tab_h, dst_tab_h, e_p_h, sgi_h, dgi_h, dsi_h, zeros_h, out_h,
          sidx, didx, dsidx, srow, drow, erow, msg, acc, sems):
        c = lax.axis_index("c")
        s = lax.axis_index("s")
        base = s * e_per_tile

        pltpu.sync_copy(sgi_h.at[c, s], sidx)
        pltpu.sync_copy(dgi_h.at[c, s], didx)
        pltpu.sync_copy(dsi_h.at[s], dsidx)
        pltpu.sync_copy(zeros_h.at[pl.ds(0, rpt_z)],
                        acc.at[pl.ds(s * rpt_z, rpt_z)])
        plsc.subcore_barrier()

        def issue(j, b):
            pltpu.async_copy(src_tab_h.at[sidx.at[j]], srow.at[b], sems[b])
            pltpu.async_copy(dst_tab_h.at[didx.at[j]], drow.at[b], sems[2 + b])
            pltpu.async_copy(e_p_h.at[c, pl.ds(base + j * _CH, _CH)],
                             erow.at[b], sems[4 + b])

        def wait(j, b):
            pltpu.make_async_copy(src_tab_h.at[sidx.at[j]], srow.at[b],
                                  sems[b]).wait()
            pltpu.make_async_copy(dst_tab_h.at[didx.at[j]], drow.at[b],
                                  sems[2 + b]).wait()
            pltpu.make_async_copy(e_p_h.at[c, pl.ds(base + j * _CH, _CH)],
                                  erow.at[b], sems[4 + b]).wait()

        def compute(j, b):
            @plsc.parallel_loop(0, _CH, unroll=2)
            def row(i):
                for jj in range(8):
                    sl = pl.ds(16 * jj, 16)
                    msg[b, i, sl] = jnp.maximum(
                        srow[b, i, sl] + drow[b, i, sl] + erow[b, i, sl], 0.0)

            pltpu.sync_copy(msg.at[b], acc.at[dsidx.at[j]], add=True)

        if dbuf:
            issue(0, 0)

            def pair(t, _):
                j0 = 2 * t
                j1 = j0 + 1
                issue(j1, 1)
                wait(j0, 0)
                compute(j0, 0)
                issue(jnp.minimum(j0 + 2, n_chunks - 1), 0)
                wait(j1, 1)
                compute(j1, 1)
                return _

            lax.fori_loop(0, n_chunks // 2, pair, None)
            wait(n_chunks - 1, 0)
        else:
            def chunk(j, _):
                issue(j, 0)
                wait(j, 0)
                compute(j, 0)
                return _

            lax.fori_loop(0, n_chunks, chunk, None)
        plsc.subcore_barrier()
        pltpu.sync_copy(acc.at[pl.ds(s * rpt_o, rpt_o)],
                        out_h.at[c, pl.ds(s * rpt_o, rpt_o)])

    return k(src_tab, dst_tab, e_p, sgi, dgi, dsi, zeros)


def _counts_sc(dsi_list, aug, zeros, n_accs, n_outs):
    n_sp = max(n_accs)
    mesh = plsc.VectorSubcoreMesh(core_axis_name="c", subcore_axis_name="s")
    max_nch = max(d.shape[2] for d in dsi_list)

    @functools.partial(
        pl.kernel,
        out_type=tuple(jax.ShapeDtypeStruct((2, n, 128), jnp.float32)
                       for n in n_outs),
        mesh=mesh,
        scratch_types=[
            pltpu.VMEM((max_nch, _CH), jnp.int32),
            pltpu.VMEM((_CH, 128), jnp.float32),
            pltpu.VMEM_SHARED((n_sp, 128), jnp.float32),
        ],
    )
    def k(*refs):
        nl = len(dsi_list)
        dsis = refs[0:nl]
        aug_h, zeros_h = refs[nl], refs[nl + 1]
        outs = refs[nl + 2:2 * nl + 2]
        dsidx, augv, acc = refs[2 * nl + 2:]
        c = lax.axis_index("c")
        s = lax.axis_index("s")
        pltpu.sync_copy(aug_h, augv)
        for l in range(nl):
            n_acc, n_out = n_accs[l], n_outs[l]
            nch = dsi_list[l].shape[2]
            rpt_z = n_acc // _NS_T
            rpt_o = n_out // _NS_T
            pltpu.sync_copy(zeros_h.at[pl.ds(0, rpt_z)],
                            acc.at[pl.ds(s * rpt_z, rpt_z)])
            plsc.subcore_barrier()
            pltpu.sync_copy(dsis[l].at[c, s], dsidx.at[pl.ds(0, nch)])

            def chunk(j, _):
                pltpu.sync_copy(augv, acc.at[dsidx.at[j]], add=True)
                return _

            lax.fori_loop(0, nch, chunk, None)
            plsc.subcore_barrier()
            pltpu.sync_copy(acc.at[pl.ds(s * rpt_o, rpt_o)],
                            outs[l].at[c, pl.ds(s * rpt_o, rpt_o)])
            plsc.subcore_barrier()

    return k(*dsi_list, aug, zeros)


def kernel(X, mesh_feat, mesh_ei, g2m_src, g2m_dst, m2g_src, m2g_dst,
           e_mm, e_g2m, e_m2g, params):
    p = params
    f32 = jnp.float32

    Xt = jnp.transpose(X[:, 0:10, :], (0, 2, 1)).reshape(N_GRID, T * 10)
    Xt = jnp.pad(Xt, ((0, N_GP - N_GRID), (0, 0)))
    X_spa = jnp.pad(X[:, 10:14, -1], ((0, N_GP - N_GRID), (0, 0)))

    w_ihT = p['gru_W_ih'].T.astype(f32)
    w_hhT = p['gru_W_hh'].T.astype(f32)

    def msg_split(w):
        return w[0:HID], w[HID:2 * HID], w[2 * HID:3 * HID]

    (g2m_w1, g2m_b1), (g2m_w2, g2m_b2) = p['g2m_msg']
    g2m_w1s, g2m_w1d, g2m_w1e = msg_split(g2m_w1)
    (m2g_w1, m2g_b1), (m2g_w2, m2g_b2) = p['m2g_msg']
    m2g_w1s, m2g_w1d, m2g_w1e = msg_split(m2g_w1)
    proc_w1s, proc_w1d, proc_w1e, proc_w2, proc_b2, proc_b1 = [], [], [], [], [], []
    for l in range(N_PROC):
        (w1, b1), (w2, b2) = p['proc'][l]
        s, d, e = msg_split(w1)
        proc_w1s.append(s); proc_w1d.append(d); proc_w1e.append(e)
        proc_w2.append(w2); proc_b2.append(b2); proc_b1.append(b1)

    (eg_w1, eg_b1), (eg_w2, eg_b2) = p['eg2m_enc']
    eg_w2f = eg_w2 @ g2m_w1e
    eg_b2f = eg_b2 @ g2m_w1e + g2m_b1
    (em_w1, em_b1), (em_w2, em_b2) = p['emm_enc']
    em_w2f = jnp.concatenate([em_w2 @ proc_w1e[l] for l in range(N_PROC)], axis=1)
    em_b2f = jnp.concatenate([em_b2 @ proc_w1e[l] + proc_b1[l]
                              for l in range(N_PROC)])
    (e2_w1, e2_b1), (e2_w2, e2_b2) = p['em2g_enc']
    e2_w2f = e2_w2 @ m2g_w1e
    e2_b2f = e2_b2 @ m2g_w1e + m2g_b1

    def prep_edges(src, dst, e_feat, dummy_dst, n_src_tab, n_dst_tab):
        E = src.shape[0]
        Ep = ((E + 2047) // 2048) * 2048
        src = jnp.pad(src.astype(jnp.int32), (0, Ep - E))
        dst = jnp.pad(dst.astype(jnp.int32), (0, Ep - E),
                      constant_values=dummy_dst)
        e_feat = jnp.pad(e_feat.astype(f32), ((0, Ep - E), (0, 0)))
        nch = Ep // (_NS_T * _CH)
        sgi = jnp.stack([src, src + n_src_tab]).reshape(2, _NS_T, nch, _CH)
        dcl = jnp.minimum(dst, n_dst_tab - 1)
        dgi = jnp.stack([dcl, dcl + n_dst_tab]).reshape(2, _NS_T, nch, _CH)
        dsi = dst.reshape(_NS_T, nch, _CH)
        dsi_h = dst.reshape(2, _NS_T, nch // 2, _CH)
        return (sgi, dgi, dsi), dsi_h, e_feat

    g2m_idx, g2m_dsih, e_g2m_p = prep_edges(g2m_src, g2m_dst, e_g2m, N_MESH,
                                            N_GP, N_MESH)
    mm_idx, mm_dsih, e_mm_p = prep_edges(mesh_ei[0], mesh_ei[1], e_mm, N_MESH,
                                         N_MESH, N_MESH)
    m2g_idx, m2g_dsih, e_m2g_p = prep_edges(m2g_src, m2g_dst, e_m2g, N_GRID,
                                            N_MESH, N_GP)

    N_ACC_M = N_MESH + 128
    N_ACC_G = N_GP
    zeros = jnp.zeros((N_GP // _NS_T, 128), f32)
    aug = jnp.zeros((_CH, 128), f32).at[:, 0].set(1.0)
    cnt_g2m, cnt_mm, cnt_m2g = _counts_sc(
        [g2m_dsih, mm_dsih, m2g_dsih], aug, zeros,
        [N_ACC_M, N_ACC_M, N_ACC_G], [N_MESH, N_MESH, N_GP])

    h = _gru_call(Xt, w_ihT, w_hhT, p['gru_b_ih'], p['gru_b_hh'])
    xgc = _bn_concat_call(h, X_spa, p['bn_g'], p['bn_b'], p['spa_g'], p['spa_b'])

    (gw1, gb1), (gw2, gb2) = p['grid_enc']
    grid, gtabs = _mlp2_call(
        xgc, gw1, gb1, gw2, gb2, ln=p['grid_ln'],
        proj_w=jnp.concatenate([g2m_w1s, m2g_w1d], axis=1))
    gsrc_tab = gtabs[0:2].reshape(2 * N_GP, 128)
    gdst_tab = gtabs[2:4].reshape(2 * N_GP, 128)
    (mw1, mb1), (mw2, mb2) = p['mesh_enc']
    mesh, mtab = _mlp2_call(mesh_feat.astype(f32), mw1, mb1, mw2, mb2,
                            ln=p['mesh_ln'], proj_w=g2m_w1d)
    mdst_tab = mtab.reshape(2 * N_MESH, 128)

    eg2m_p = _mlp2_call(e_g2m_p, eg_w1, eg_b1, eg_w2f, eg_b2f, split_out=True)
    emm_p = _mlp2_call(e_mm_p, em_w1, em_b1, em_w2f, em_b2f, split_out=True)
    em2g_p = _mlp2_call(e_m2g_p, e2_w1, e2_b1, e2_w2f, e2_b2f, split_out=True)

    acc = _edge_stage_sc(gsrc_tab, mdst_tab, eg2m_p, *g2m_idx, zeros,
                         N_ACC_M, N_MESH)
    mesh, sd = _post_stage_call(
        acc, cnt_g2m, mesh, g2m_w2, g2m_b2,
        proj_w=jnp.concatenate([proc_w1s[0], proc_w1d[0]], axis=1))

    for l in range(N_PROC):
        s_tab = sd[0:2].reshape(2 * N_MESH, 128)
        d_tab = sd[2:4].reshape(2 * N_MESH, 128)
        acc = _edge_stage_sc(s_tab, d_tab, emm_p[2 * l:2 * l + 2], *mm_idx,
                             zeros, N_ACC_M, N_MESH)
        if l + 1 < N_PROC:
            pw = jnp.concatenate([proc_w1s[l + 1], proc_w1d[l + 1]], axis=1)
        else:
            pw = m2g_w1s
        mesh, sd = _post_stage_call(acc, cnt_mm, mesh, proc_w2[l], proc_b2[l],
                                    ln=p['proc_ln'][l], proj_w=pw)

    msrc_tab = sd[0:2].reshape(2 * N_MESH, 128)
    acc = _edge_stage_sc(msrc_tab, gdst_tab, em2g_p, *m2g_idx, zeros,
                         N_ACC_G, N_GP, dbuf=False)
    grid = _post_stage_call(acc, cnt_m2g, grid, m2g_w2, m2g_b2)

    output, logits, hidden = _decoder_call(grid, p)
    return (output[0:N_GRID, 0:4], logits[0:N_GRID, 0:4], hidden[0:N_GRID])
